# Initial kernel scaffold; baseline (speedup 1.0000x reference)
import jax
import jax.numpy as jnp
from jax.experimental import pallas as pl


def kernel(feature_center, feature_metapath, segment_ids, type_weight, W1, b1, W2, b2, W3, b3, Wc, bc):
    raise NotImplementedError("write your pallas kernel here")


# speedup vs baseline: 2.5121x; 2.5121x over previous
import functools

import jax
import jax.numpy as jnp
from jax import lax
from jax.experimental import pallas as pl
from jax.experimental.pallas import tpu as pltpu
from jax.experimental.pallas import tpu_sc as plsc


def _mlp_body(fm_ref, w1_ref, b1_ref, w2_ref, b2_ref, w3_ref, b3_ref, out_ref):
    h = jnp.dot(fm_ref[...], w1_ref[...], preferred_element_type=jnp.float32)
    h = jnp.maximum(h + b1_ref[...], 0.0)
    h = jnp.dot(h, w2_ref[...], preferred_element_type=jnp.float32)
    h = jnp.maximum(h + b2_ref[...], 0.0)
    h = jnp.dot(h, w3_ref[...], preferred_element_type=jnp.float32)
    out_ref[...] = h + b3_ref[...]


def _mlp(fm, w1, b1, w2, b2, w3, b3, block_rows=1600):
    e, d = fm.shape
    hdim = w1.shape[1]
    p = w3.shape[1]
    assert e % block_rows == 0
    grid = (e // block_rows,)
    rep = lambda i: (0, 0)
    return pl.pallas_call(
        _mlp_body,
        grid=grid,
        in_specs=[
            pl.BlockSpec((block_rows, d), lambda i: (i, 0)),
            pl.BlockSpec((d, hdim), rep),
            pl.BlockSpec((1, hdim), rep),
            pl.BlockSpec((hdim, hdim), rep),
            pl.BlockSpec((1, hdim), rep),
            pl.BlockSpec((hdim, p), rep),
            pl.BlockSpec((1, p), rep),
        ],
        out_specs=pl.BlockSpec((block_rows, p), lambda i: (i, 0)),
        out_shape=jax.ShapeDtypeStruct((e, p), jnp.float32),
        compiler_params=pltpu.CompilerParams(
            dimension_semantics=("arbitrary",),
        ),
    )(fm, w1, b1.reshape(1, hdim), w2, b2.reshape(1, hdim),
      w3, b3.reshape(1, p))


_NC = 2
_NS = 16
_CHUNK = 80
_CH = 128


def _segment_sum_sc(inj, ids, n):
    e, p = inj.shape
    assert p == _NC * _CH
    per_tile = e // _NS
    n_chunks = per_tile // _CHUNK
    assert per_tile % _CHUNK == 0 and e % _NS == 0
    wb_tiles = 10
    out_rows = n // wb_tiles
    zrows = 40
    assert n % wb_tiles == 0 and out_rows % zrows == 0 and zrows % 8 == 0

    ids3d = ids.reshape(_NS, n_chunks, _CHUNK)
    mesh = plsc.VectorSubcoreMesh(core_axis_name="c", subcore_axis_name="s")

    @functools.partial(
        pl.kernel,
        mesh=mesh,
        out_type=jax.ShapeDtypeStruct((n, p), jnp.float32),
        scratch_types=[
            pltpu.VMEM((n_chunks, _CHUNK), jnp.int32),
            pltpu.VMEM((_CHUNK, _CH), jnp.float32),
            pltpu.VMEM((zrows, _CH), jnp.float32),
            pltpu.VMEM_SHARED((n, _CH), jnp.float32),
        ],
    )
    def segsum(inj_hbm, ids_hbm, out_hbm, ids_v, dbuf, zbuf, acc):
        c = lax.axis_index("c")
        s = lax.axis_index("s")
        col0 = c * _CH

        zeros16 = jnp.zeros((16,), jnp.float32)

        def zb(i, carry):
            r = i // (_CH // 16)
            k = i % (_CH // 16)
            zbuf[r, pl.ds(k * 16, 16)] = zeros16
            return carry

        lax.fori_loop(0, zrows * (_CH // 16), zb, 0)

        @pl.when(s < wb_tiles)
        def _zero():
            def zc(k, carry):
                pltpu.sync_copy(
                    zbuf, acc.at[pl.ds(s * out_rows + k * zrows, zrows), :])
                return carry

            lax.fori_loop(0, out_rows // zrows, zc, 0)

        plsc.subcore_barrier()

        pltpu.sync_copy(ids_hbm.at[s], ids_v)

        def body(j, carry):
            r0 = s * per_tile + j * _CHUNK
            pltpu.sync_copy(inj_hbm.at[pl.ds(r0, _CHUNK), pl.ds(col0, _CH)],
                            dbuf)
            pltpu.sync_copy(dbuf, acc.at[ids_v.at[j]], add=True)
            return carry

        lax.fori_loop(0, n_chunks, body, 0)
        plsc.subcore_barrier()

        @pl.when(s < wb_tiles)
        def _writeback():
            pltpu.sync_copy(
                acc.at[pl.ds(s * out_rows, out_rows), :],
                out_hbm.at[pl.ds(s * out_rows, out_rows), pl.ds(col0, _CH)])

    return segsum(inj, ids3d)


def _fin_body(fc_ref, tw_ref, agg_ref, wc_ref, bc_ref, pre_ref):
    center = jnp.dot(fc_ref[...], tw_ref[...],
                     preferred_element_type=jnp.float32)
    pre_ref[...] = jnp.dot(center + agg_ref[...], wc_ref[...],
                           preferred_element_type=jnp.float32) + bc_ref[...]


def _final(fc, tw, agg, wc, bc, block_rows=1000):
    n, d = fc.shape
    p = tw.shape[1]
    s = wc.shape[1]
    assert n % block_rows == 0
    grid = (n // block_rows,)
    rep = lambda i: (0, 0)
    return pl.pallas_call(
        _fin_body,
        grid=grid,
        in_specs=[
            pl.BlockSpec((block_rows, d), lambda i: (i, 0)),
            pl.BlockSpec((d, p), rep),
            pl.BlockSpec((block_rows, p), lambda i: (i, 0)),
            pl.BlockSpec((p, s), rep),
            pl.BlockSpec((1, s), rep),
        ],
        out_specs=pl.BlockSpec((block_rows, s), lambda i: (i, 0)),
        out_shape=jax.ShapeDtypeStruct((n, s), jnp.float32),
        compiler_params=pltpu.CompilerParams(
            dimension_semantics=("arbitrary",),
        ),
    )(fc, tw, agg, wc, bc.reshape(1, s))


def kernel(feature_center, feature_metapath, segment_ids, type_weight,
           W1, b1, W2, b2, W3, b3, Wc, bc):
    n = feature_center.shape[0]
    inj = _mlp(feature_metapath, W1, b1, W2, b2, W3, b3)
    agg = _segment_sum_sc(inj, segment_ids, n)
    pre_embed = _final(feature_center, type_weight, agg, Wc, bc)
    return (pre_embed, agg)


# speedup vs baseline: 3.1006x; 1.2343x over previous
import functools

import jax
import jax.numpy as jnp
from jax import lax
from jax.experimental import pallas as pl
from jax.experimental.pallas import tpu as pltpu
from jax.experimental.pallas import tpu_sc as plsc


def _mlp_body(fm_ref, w1_ref, b1_ref, w2_ref, b2_ref, w3_ref, b3_ref, out_ref):
    h = jnp.dot(fm_ref[...], w1_ref[...], preferred_element_type=jnp.float32)
    h = jnp.maximum(h + b1_ref[...], 0.0)
    h = jnp.dot(h, w2_ref[...], preferred_element_type=jnp.float32)
    h = jnp.maximum(h + b2_ref[...], 0.0)
    h = jnp.dot(h, w3_ref[...], preferred_element_type=jnp.float32)
    out_ref[...] = h + b3_ref[...]


def _mlp(fm, w1, b1, w2, b2, w3, b3, block_rows=1600):
    e, d = fm.shape
    hdim = w1.shape[1]
    p = w3.shape[1]
    assert e % block_rows == 0
    grid = (e // block_rows,)
    rep = lambda i: (0, 0)
    return pl.pallas_call(
        _mlp_body,
        grid=grid,
        in_specs=[
            pl.BlockSpec((block_rows, d), lambda i: (i, 0)),
            pl.BlockSpec((d, hdim), rep),
            pl.BlockSpec((1, hdim), rep),
            pl.BlockSpec((hdim, hdim), rep),
            pl.BlockSpec((1, hdim), rep),
            pl.BlockSpec((hdim, p), rep),
            pl.BlockSpec((1, p), rep),
        ],
        out_specs=pl.BlockSpec((block_rows, p), lambda i: (i, 0)),
        out_shape=jax.ShapeDtypeStruct((e, p), jnp.float32),
        compiler_params=pltpu.CompilerParams(
            dimension_semantics=("arbitrary",),
        ),
    )(fm, w1, b1.reshape(1, hdim), w2, b2.reshape(1, hdim),
      w3, b3.reshape(1, p))


_NC = 2
_NS = 16
_CHUNK = 80
_CH = 128


def _segment_sum_sc(inj, ids, n):
    e, p = inj.shape
    assert p == _NC * _CH
    per_tile = e // _NS
    n_chunks = per_tile // _CHUNK
    assert per_tile % _CHUNK == 0 and e % _NS == 0
    wb_tiles = 10
    out_rows = n // wb_tiles
    zrows = 40
    assert n % wb_tiles == 0 and out_rows % zrows == 0 and zrows % 8 == 0

    ids3d = ids.reshape(_NS, n_chunks, _CHUNK)
    mesh = plsc.VectorSubcoreMesh(core_axis_name="c", subcore_axis_name="s")

    @functools.partial(
        pl.kernel,
        mesh=mesh,
        out_type=jax.ShapeDtypeStruct((n, p), jnp.float32),
        scratch_types=[
            pltpu.VMEM((n_chunks, _CHUNK), jnp.int32),
            pltpu.VMEM((2, _CHUNK, _CH), jnp.float32),
            pltpu.VMEM((zrows, _CH), jnp.float32),
            pltpu.VMEM_SHARED((n, _CH), jnp.float32),
            pltpu.SemaphoreType.DMA,
            pltpu.SemaphoreType.DMA,
        ],
    )
    def segsum(inj_hbm, ids_hbm, out_hbm, ids_v, dbuf, zbuf, acc, sem0, sem1):
        c = lax.axis_index("c")
        s = lax.axis_index("s")
        col0 = c * _CH
        base = s * per_tile

        ids_cp = pltpu.async_copy(ids_hbm.at[s], ids_v, sem0)

        zeros16 = jnp.zeros((16,), jnp.float32)

        def zb(i, carry):
            r = i // (_CH // 16)
            k = i % (_CH // 16)
            zbuf[r, pl.ds(k * 16, 16)] = zeros16
            return carry

        lax.fori_loop(0, zrows * (_CH // 16), zb, 0)

        @pl.when(s < wb_tiles)
        def _zero():
            def zc(k, carry):
                pltpu.sync_copy(
                    zbuf, acc.at[pl.ds(s * out_rows + k * zrows, zrows), :])
                return carry

            lax.fori_loop(0, out_rows // zrows, zc, 0)

        ids_cp.wait()
        plsc.subcore_barrier()

        def _src(j):
            return inj_hbm.at[pl.ds(base + j * _CHUNK, _CHUNK),
                              pl.ds(col0, _CH)]

        sems = (sem0, sem1)

        def _start(j, b):
            pltpu.async_copy(_src(j), dbuf.at[b], sems[b])

        def _finish(j, b):
            pltpu.make_async_copy(_src(j), dbuf.at[b], sems[b]).wait()
            pltpu.sync_copy(dbuf.at[b], acc.at[ids_v.at[j]], add=True)

        assert n_chunks % 2 == 1
        _start(0, 0)

        def body(i, carry):
            j0 = 2 * i
            _start(j0 + 1, 1)
            _finish(j0, 0)

            @pl.when(j0 + 2 < n_chunks)
            def _():
                _start(j0 + 2, 0)

            _finish(j0 + 1, 1)
            return carry

        lax.fori_loop(0, n_chunks // 2, body, 0)
        _finish(n_chunks - 1, 0)
        plsc.subcore_barrier()

        @pl.when(s < wb_tiles)
        def _writeback():
            pltpu.sync_copy(
                acc.at[pl.ds(s * out_rows, out_rows), :],
                out_hbm.at[pl.ds(s * out_rows, out_rows), pl.ds(col0, _CH)])

    return segsum(inj, ids3d)


def _fin_body(fc_ref, tw_ref, agg_ref, wc_ref, bc_ref, pre_ref):
    center = jnp.dot(fc_ref[...], tw_ref[...],
                     preferred_element_type=jnp.float32)
    pre_ref[...] = jnp.dot(center + agg_ref[...], wc_ref[...],
                           preferred_element_type=jnp.float32) + bc_ref[...]


def _final(fc, tw, agg, wc, bc, block_rows=1000):
    n, d = fc.shape
    p = tw.shape[1]
    s = wc.shape[1]
    assert n % block_rows == 0
    grid = (n // block_rows,)
    rep = lambda i: (0, 0)
    return pl.pallas_call(
        _fin_body,
        grid=grid,
        in_specs=[
            pl.BlockSpec((block_rows, d), lambda i: (i, 0)),
            pl.BlockSpec((d, p), rep),
            pl.BlockSpec((block_rows, p), lambda i: (i, 0)),
            pl.BlockSpec((p, s), rep),
            pl.BlockSpec((1, s), rep),
        ],
        out_specs=pl.BlockSpec((block_rows, s), lambda i: (i, 0)),
        out_shape=jax.ShapeDtypeStruct((n, s), jnp.float32),
        compiler_params=pltpu.CompilerParams(
            dimension_semantics=("arbitrary",),
        ),
    )(fc, tw, agg, wc, bc.reshape(1, s))


def kernel(feature_center, feature_metapath, segment_ids, type_weight,
           W1, b1, W2, b2, W3, b3, Wc, bc):
    n = feature_center.shape[0]
    inj = _mlp(feature_metapath, W1, b1, W2, b2, W3, b3)
    agg = _segment_sum_sc(inj, segment_ids, n)
    pre_embed = _final(feature_center, type_weight, agg, Wc, bc)
    return (pre_embed, agg)
